# baseline (speedup 1.0000x reference)
import functools

import jax
import jax.numpy as jnp
from jax import lax
from jax.experimental import pallas as pl
from jax.experimental.pallas import tpu as pltpu
from jax.experimental.pallas import tpu_sc as plsc

N_OBJECTS = 1000000
EMBEDDING_DIM = 64
BATCH = 16384

_info = plsc.get_sparse_core_info()
_NC, _NS = _info.num_cores, _info.num_subcores
_NW = _NC * _NS
_B_PER_W = BATCH // _NW
_CHUNK = 128
_N_CHUNKS = _B_PER_W // _CHUNK


def _embedding_kernel(idx_hbm, table_hbm, out_hbm, idx_v, rows_v, sem):
    wid = lax.axis_index("s") * _NC + lax.axis_index("c")
    base = wid * _N_CHUNKS

    pltpu.sync_copy(idx_hbm.at[pl.ds(base, _N_CHUNKS)], idx_v)

    copies = [
        pltpu.async_copy(
            table_hbm.at[idx_v.at[g]],
            rows_v.at[pl.ds(g * _CHUNK, _CHUNK)],
            sem)
        for g in range(_N_CHUNKS)
    ]
    for c in copies:
        c.wait()

    pltpu.sync_copy(rows_v, out_hbm.at[pl.ds(wid * _B_PER_W, _B_PER_W)])


@jax.jit
def kernel(o_idxs, table):
    mesh = plsc.VectorSubcoreMesh(core_axis_name="c", subcore_axis_name="s")
    run = functools.partial(
        pl.kernel,
        mesh=mesh,
        compiler_params=pltpu.CompilerParams(use_tc_tiling_on_sc=False),
        out_type=jax.ShapeDtypeStruct((BATCH, EMBEDDING_DIM), jnp.float32),
        scratch_types=[
            pltpu.VMEM((_N_CHUNKS, _CHUNK), jnp.int32),
            pltpu.VMEM((_B_PER_W, EMBEDDING_DIM), jnp.float32),
            pltpu.SemaphoreType.DMA,
        ],
    )(_embedding_kernel)
    idx2d = o_idxs.astype(jnp.int32).reshape(BATCH // _CHUNK, _CHUNK)
    return run(idx2d, table)


# speedup vs baseline: 1.1217x; 1.1217x over previous
import functools

import jax
import jax.numpy as jnp
from jax import lax
from jax.experimental import pallas as pl
from jax.experimental.pallas import tpu as pltpu
from jax.experimental.pallas import tpu_sc as plsc

N_OBJECTS = 1000000
EMBEDDING_DIM = 64
BATCH = 16384
WIDE = 128

_info = plsc.get_sparse_core_info()
_NC, _NS = _info.num_cores, _info.num_subcores
_NW = _NC * _NS
_B_PER_W = BATCH // _NW
_CHUNK = 128
_N_CHUNKS = _B_PER_W // _CHUNK


def _gather_kernel(idx_hbm, wide_hbm, out_hbm, idx_v, rows_v, sem):
    wid = lax.axis_index("s") * _NC + lax.axis_index("c")

    pltpu.sync_copy(idx_hbm.at[pl.ds(wid * _N_CHUNKS, _N_CHUNKS)], idx_v)

    copies = [
        pltpu.async_copy(
            wide_hbm.at[idx_v.at[g]],
            rows_v.at[pl.ds(g * _CHUNK, _CHUNK)],
            sem)
        for g in range(_N_CHUNKS)
    ]
    for c in copies:
        c.wait()

    pltpu.sync_copy(rows_v, out_hbm.at[pl.ds(wid * _B_PER_W, _B_PER_W)])


@jax.jit
def kernel(o_idxs, table):
    mesh = plsc.VectorSubcoreMesh(core_axis_name="c", subcore_axis_name="s")
    gather = pl.kernel(
        _gather_kernel,
        mesh=mesh,
        compiler_params=pltpu.CompilerParams(use_tc_tiling_on_sc=False),
        out_type=jax.ShapeDtypeStruct((BATCH, WIDE), jnp.float32),
        scratch_types=[
            pltpu.VMEM((_N_CHUNKS, _CHUNK), jnp.int32),
            pltpu.VMEM((_B_PER_W, WIDE), jnp.float32),
            pltpu.SemaphoreType.DMA,
        ],
    )
    wide = jnp.pad(table, ((0, 0), (0, WIDE - EMBEDDING_DIM)))
    idx2d = o_idxs.astype(jnp.int32).reshape(BATCH // _CHUNK, _CHUNK)
    return gather(idx2d, wide)[:, :EMBEDDING_DIM]
